# baseline (speedup 1.0000x reference)
import jax
import jax.numpy as jnp
from jax.experimental import pallas as pl
from jax.experimental.pallas import tpu as pltpu

NV_PREV = 10242
N_STRIPES = 4

_bounds = [round(NV_PREV * i / N_STRIPES) for i in range(N_STRIPES + 1)]
_STRIPES = [(_bounds[i], _bounds[i + 1] - _bounds[i]) for i in range(N_STRIPES)]


def _copy_body(x_ref, o_ref, sems):
    copies = []
    for i, (start, size) in enumerate(_STRIPES):
        cp = pltpu.make_async_copy(
            x_ref.at[pl.ds(start, size)],
            o_ref.at[pl.ds(start, size)],
            sems.at[i],
        )
        cp.start()
        copies.append(cp)
    for cp in copies:
        cp.wait()


def kernel(x):
    return pl.pallas_call(
        _copy_body,
        out_shape=jax.ShapeDtypeStruct((NV_PREV,) + x.shape[1:], x.dtype),
        in_specs=[pl.BlockSpec(memory_space=pl.ANY)],
        out_specs=pl.BlockSpec(memory_space=pl.ANY),
        scratch_shapes=[pltpu.SemaphoreType.DMA((N_STRIPES,))],
    )(x)


# speedup vs baseline: 2.0231x; 2.0231x over previous
import jax
import jax.numpy as jnp
from jax.experimental import pallas as pl
from jax.experimental.pallas import tpu as pltpu

NV_PREV = 10242
N_STRIPES = 4

_bounds = [round(NV_PREV * i / N_STRIPES) for i in range(N_STRIPES + 1)]
_STRIPES = [(_bounds[i], _bounds[i + 1] - _bounds[i]) for i in range(N_STRIPES)]


def _copy_body(x_ref, o_ref, sems):
    copies = []
    for i, (start, size) in enumerate(_STRIPES):
        cp = pltpu.make_async_copy(
            x_ref.at[pl.ds(start, size)],
            o_ref.at[pl.ds(start, size)],
            sems.at[i],
        )
        cp.start()
        copies.append(cp)
    for cp in copies:
        cp.wait()


def kernel(x):
    n, a, b, c = x.shape
    x2 = x.reshape(n, 8, 128)
    out2 = pl.pallas_call(
        _copy_body,
        out_shape=jax.ShapeDtypeStruct((NV_PREV, 8, 128), x.dtype),
        in_specs=[pl.BlockSpec(memory_space=pl.ANY)],
        out_specs=pl.BlockSpec(memory_space=pl.ANY),
        scratch_shapes=[pltpu.SemaphoreType.DMA((N_STRIPES,))],
    )(x2)
    return out2.reshape(NV_PREV, a, b, c)


# speedup vs baseline: 13.9001x; 6.8708x over previous
import jax
import jax.numpy as jnp
from jax.experimental import pallas as pl
from jax.experimental.pallas import tpu as pltpu

NV_PREV = 10242
BLOCK = 1138


def _copy_body(x_ref, o_ref):
    o_ref[...] = x_ref[...]


def kernel(x):
    n, a, b, c = x.shape
    x2 = x.reshape(n, 8, 128)
    out2 = pl.pallas_call(
        _copy_body,
        grid=(NV_PREV // BLOCK,),
        in_specs=[pl.BlockSpec((BLOCK, 8, 128), lambda i: (i, 0, 0))],
        out_specs=pl.BlockSpec((BLOCK, 8, 128), lambda i: (i, 0, 0)),
        out_shape=jax.ShapeDtypeStruct((NV_PREV, 8, 128), x.dtype),
    )(x2)
    return out2.reshape(NV_PREV, a, b, c)
